# baseline (speedup 1.0000x reference)
import functools

import jax
import jax.numpy as jnp
from jax import lax
from jax.experimental import pallas as pl
from jax.experimental.pallas import tpu as pltpu
from jax.experimental.pallas import tpu_sc as plsc

MAX_RANKS = 200
BATCH = 16384
LIST_LEN = 200

NUM_CORES = 2
NUM_SUBCORES = 16
LANES = 16

NW = NUM_CORES * NUM_SUBCORES
COLS_PER = BATCH // NW
ROW_CHUNK = 40
N_CHUNKS = LIST_LEN // ROW_CHUNK
COL_VECS = COLS_PER // LANES

_mesh = plsc.VectorSubcoreMesh(
    core_axis_name="c",
    subcore_axis_name="s",
    num_cores=NUM_CORES,
    num_subcores=NUM_SUBCORES,
)


@functools.partial(
    pl.kernel,
    out_type=jax.ShapeDtypeStruct((LIST_LEN, 1, BATCH), jnp.float32),
    mesh=_mesh,
    compiler_params=pltpu.CompilerParams(needs_layout_passes=False),
    scratch_types=[
        pltpu.VMEM((MAX_RANKS + 8,), jnp.float32),
        pltpu.VMEM((MAX_RANKS,), jnp.float32),
        pltpu.VMEM((ROW_CHUNK, COLS_PER), jnp.int32),
        pltpu.VMEM((ROW_CHUNK, COLS_PER), jnp.int32),
        pltpu.VMEM((ROW_CHUNK, COLS_PER), jnp.float32),
        pltpu.VMEM((ROW_CHUNK, COLS_PER), jnp.float32),
        pltpu.SemaphoreType.DMA,
        pltpu.SemaphoreType.DMA,
    ],
)
def _positional_bias_kernel(
    idx_hbm, w_hbm, out_hbm, t_v, w_v, idx_v0, idx_v1, out_v0, out_v1,
    in_sem, out_sem,
):
    wid = lax.axis_index("s") * NUM_CORES + lax.axis_index("c")
    col0 = wid * COLS_PER

    pltpu.sync_copy(w_hbm, w_v)
    for k in range(0, MAX_RANKS, LANES):
        ii = lax.iota(jnp.int32, LANES) + (k - 1)
        safe = jnp.clip(ii, 0, MAX_RANKS - 1)
        g = plsc.load_gather(w_v, [safe])
        t_v[pl.ds(k, LANES)] = jnp.where(ii >= 0, g, 0.0)

    idx_bufs = (idx_v0, idx_v1)
    out_bufs = (out_v0, out_v1)

    in_copies = [None] * N_CHUNKS
    out_copies = [None] * N_CHUNKS

    in_copies[0] = pltpu.async_copy(
        idx_hbm.at[pl.ds(0, ROW_CHUNK), pl.ds(col0, COLS_PER)],
        idx_bufs[0], in_sem)

    for c in range(N_CHUNKS):
        idx_v = idx_bufs[c % 2]
        out_v = out_bufs[c % 2]

        if c + 1 < N_CHUNKS:
            in_copies[c + 1] = pltpu.async_copy(
                idx_hbm.at[pl.ds((c + 1) * ROW_CHUNK, ROW_CHUNK),
                           pl.ds(col0, COLS_PER)],
                idx_bufs[(c + 1) % 2], in_sem)

        in_copies[c].wait()
        if c >= 2:
            out_copies[c - 2].wait()

        @plsc.parallel_loop(0, ROW_CHUNK, step=1, unroll=10)
        def _gather_body(r):
            for cc in range(COL_VECS):
                raw = idx_v[r, pl.ds(cc * LANES, LANES)]
                out_v[r, pl.ds(cc * LANES, LANES)] = plsc.load_gather(
                    t_v, [raw])

        out_copies[c] = pltpu.async_copy(
            out_v,
            out_hbm.at[pl.ds(c * ROW_CHUNK, ROW_CHUNK), 0,
                       pl.ds(col0, COLS_PER)],
            out_sem)

    out_copies[N_CHUNKS - 2].wait()
    out_copies[N_CHUNKS - 1].wait()


def kernel(inputs, W):
    out3 = _positional_bias_kernel(inputs.T, W.reshape(-1))
    return out3.transpose(2, 0, 1)


# speedup vs baseline: 1.1258x; 1.1258x over previous
import functools

import jax
import jax.numpy as jnp
from jax import lax
from jax.experimental import pallas as pl
from jax.experimental.pallas import tpu as pltpu
from jax.experimental.pallas import tpu_sc as plsc

MAX_RANKS = 200
BATCH = 16384
LIST_LEN = 200

NUM_CORES = 2
NUM_SUBCORES = 16
LANES = 16

NW = NUM_CORES * NUM_SUBCORES
COLS_PER = BATCH // NW
ROW_CHUNK = 40
N_CHUNKS = LIST_LEN // ROW_CHUNK
COL_VECS = COLS_PER // LANES

_mesh = plsc.VectorSubcoreMesh(
    core_axis_name="c",
    subcore_axis_name="s",
    num_cores=NUM_CORES,
    num_subcores=NUM_SUBCORES,
)


@functools.partial(
    pl.kernel,
    out_type=jax.ShapeDtypeStruct((LIST_LEN, 1, BATCH), jnp.float32),
    mesh=_mesh,
    compiler_params=pltpu.CompilerParams(needs_layout_passes=False),
    scratch_types=[
        pltpu.VMEM((MAX_RANKS + 8,), jnp.float32),
        pltpu.VMEM((MAX_RANKS,), jnp.float32),
        pltpu.VMEM((ROW_CHUNK, COLS_PER), jnp.int32),
        pltpu.VMEM((ROW_CHUNK, COLS_PER), jnp.int32),
        pltpu.VMEM((ROW_CHUNK, COLS_PER), jnp.float32),
        pltpu.VMEM((ROW_CHUNK, COLS_PER), jnp.float32),
        pltpu.SemaphoreType.DMA,
        pltpu.SemaphoreType.DMA,
    ],
)
def _positional_bias_kernel(
    idx_hbm, w_hbm, out_hbm, t_v, w_v, idx_v0, idx_v1, out_v0, out_v1,
    in_sem, out_sem,
):
    wid = lax.axis_index("s") * NUM_CORES + lax.axis_index("c")
    col0 = wid * COLS_PER

    idx_bufs = (idx_v0, idx_v1)
    out_bufs = (out_v0, out_v1)

    in_copies = [None] * N_CHUNKS
    out_copies = [None] * N_CHUNKS

    in_copies[0] = pltpu.async_copy(
        idx_hbm.at[pl.ds(0, ROW_CHUNK), pl.ds(col0, COLS_PER)],
        idx_bufs[0], in_sem)

    pltpu.sync_copy(w_hbm, w_v)
    for k in range(0, MAX_RANKS, LANES):
        ii = lax.iota(jnp.int32, LANES) + (k - 1)
        safe = jnp.clip(ii, 0, MAX_RANKS - 1)
        g = plsc.load_gather(w_v, [safe])
        t_v[pl.ds(k, LANES)] = jnp.where(ii >= 0, g, 0.0)

    for c in range(N_CHUNKS):
        idx_v = idx_bufs[c % 2]
        out_v = out_bufs[c % 2]

        if c + 1 < N_CHUNKS:
            in_copies[c + 1] = pltpu.async_copy(
                idx_hbm.at[pl.ds((c + 1) * ROW_CHUNK, ROW_CHUNK),
                           pl.ds(col0, COLS_PER)],
                idx_bufs[(c + 1) % 2], in_sem)

        in_copies[c].wait()
        if c >= 2:
            out_copies[c - 2].wait()

        @plsc.parallel_loop(0, ROW_CHUNK, step=1, unroll=8)
        def _gather_body(r):
            for cc in range(COL_VECS):
                raw = idx_v[r, pl.ds(cc * LANES, LANES)]
                out_v[r, pl.ds(cc * LANES, LANES)] = plsc.load_gather(
                    t_v, [raw])

        out_copies[c] = pltpu.async_copy(
            out_v,
            out_hbm.at[pl.ds(c * ROW_CHUNK, ROW_CHUNK), 0,
                       pl.ds(col0, COLS_PER)],
            out_sem)

    out_copies[N_CHUNKS - 2].wait()
    out_copies[N_CHUNKS - 1].wait()


def kernel(inputs, W):
    out3 = _positional_bias_kernel(inputs.T, W.reshape(-1))
    return out3.transpose(2, 0, 1)
